# baseline (speedup 1.0000x reference)
import numpy as np

import jax
import jax.numpy as jnp
from jax import lax
from jax.experimental import pallas as pl
from jax.experimental.pallas import tpu as pltpu
from jax.experimental.pallas import tpu_sc as plsc

NSUB = 16
NP0 = 192
NP1 = 448
C0TOT = NSUB * NP0
NPAD = NSUB * (NP0 + NP1)
MAXDEG = 7
NB0 = 12
NB1 = 26
CAP = NB1 * 128
D = 128
LANES = 8


def _bcast_lane(v, l):
    idx = jnp.full((16, 1), l, jnp.int32)
    dnums = lax.GatherDimensionNumbers(
        offset_dims=(), collapsed_slice_dims=(0,), start_index_map=(0,))
    return lax.gather(v, idx, dnums, (1,),
                      mode=lax.GatherScatterMode.PROMISE_IN_BOUNDS)


def _agg_body(feat, cols_p, rows_p, w_p, se, num_out, den_out,
              colsb, rowsb, wb, sev, gath0, gath1, num_acc, den_acc,
              sem0, sem1):
    c = lax.axis_index("c")
    s = lax.axis_index("s")
    wid = s * 2 + c

    pltpu.sync_copy(se.at[wid], sev)
    sevv = sev[...]
    start = sevv[0]
    end = sevv[1]
    astart = pl.multiple_of(jnp.bitwise_and(start, -8), 8)
    lo = start - astart
    hi = lo + (end - start)

    zero = jnp.zeros((16,), jnp.float32)
    iota = lax.broadcasted_iota(jnp.int32, (16,), 0)

    def tile_flow(npc, nbc, node_base):
        capc = nbc * 128
        pltpu.sync_copy(cols_p.at[pl.ds(astart, capc)],
                        colsb.at[pl.ds(0, capc)])
        pltpu.sync_copy(rows_p.at[pl.ds(astart, capc)],
                        rowsb.at[pl.ds(0, capc)])
        pltpu.sync_copy(w_p.at[pl.ds(astart, capc)], wb.at[pl.ds(0, capc)])

        def zrow(i, _):
            for u in range(16):
                num_acc[pl.ds(pl.multiple_of(i * 256 + u * 16, 16), 16)] = zero
            return 0

        lax.fori_loop(0, npc * D // 256, zrow, 0)

        def zden(i, _):
            for u in range(16):
                den_acc[pl.ds(pl.multiple_of(i * 256 + u * 16, 16), 16)] = zero
            return 0

        lax.fori_loop(0, npc * 16 // 256, zden, 0)

        def block_slice(j):
            return colsb.at[pl.ds(pl.multiple_of(j * 128, 128), 128)]

        def process_block(j, gath):
            def gbody(g, _):
                base = pl.multiple_of(g * 16, 16)
                sbase = j * 128 + base
                w_v = wb[pl.ds(sbase, 16)]
                r_v = rowsb[pl.ds(sbase, 16)]
                gidx = sbase + iota
                valid = (gidx >= lo) & (gidx < hi)
                wm_v = jnp.where(valid, w_v, 0.0)
                rl_v = jnp.clip(r_v - node_base, 0, npc - 1)
                plsc.addupdate_scatter(den_acc, [rl_v * 16 + iota], wm_v)
                ribase = rl_v * D
                for c0 in range(0, 16, 4):
                    scaled = []
                    for l in range(c0, c0 + 4):
                        w_b = _bcast_lane(wm_v, l)
                        ri = _bcast_lane(ribase, l) + iota
                        e = base + l
                        for k in range(LANES):
                            v = gath[e, pl.ds(k * 16, 16)]
                            scaled.append((ri + k * 16, v * w_b))
                    for idx, val in scaled:
                        plsc.addupdate_scatter(num_acc, [idx], val)
                return 0

            lax.fori_loop(0, 128 // 16, gbody, 0)

        pltpu.async_copy(feat.at[block_slice(0)], gath0, sem0)

        def pair_body(t, _):
            j0 = t * 2
            j1 = j0 + 1
            pltpu.make_async_copy(feat.at[block_slice(j0)], gath0, sem0).wait()
            pltpu.async_copy(feat.at[block_slice(j1)], gath1, sem1)
            process_block(j0, gath0)
            pltpu.make_async_copy(feat.at[block_slice(j1)], gath1, sem1).wait()

            @pl.when(t < nbc // 2 - 1)
            def _():
                pltpu.async_copy(feat.at[block_slice(j0 + 2)], gath0, sem0)

            process_block(j1, gath1)
            return 0

        lax.fori_loop(0, nbc // 2, pair_body, 0)

        obase = pl.multiple_of(node_base * D, 128)
        pltpu.sync_copy(num_acc.at[pl.ds(0, npc * D)],
                        num_out.at[pl.ds(obase, npc * D)])
        dbase = pl.multiple_of(node_base * 16, 16)
        pltpu.sync_copy(den_acc.at[pl.ds(0, npc * 16)],
                        den_out.at[pl.ds(dbase, npc * 16)])

    @pl.when(c == 0)
    def _():
        tile_flow(NP0, NB0, s * NP0)

    @pl.when(c == 1)
    def _():
        tile_flow(NP1, NB1, C0TOT + s * NP1)


_agg = pl.kernel(
    _agg_body,
    out_type=[jax.ShapeDtypeStruct((NPAD * D,), jnp.float32),
              jax.ShapeDtypeStruct((NPAD * 16,), jnp.float32)],
    mesh=plsc.VectorSubcoreMesh(core_axis_name="c", subcore_axis_name="s"),
    compiler_params=pltpu.CompilerParams(needs_layout_passes=False),
    scratch_types=[
        pltpu.VMEM((CAP,), jnp.int32),
        pltpu.VMEM((CAP,), jnp.int32),
        pltpu.VMEM((CAP,), jnp.float32),
        pltpu.VMEM((16,), jnp.int32),
        pltpu.VMEM((128, D), jnp.float32),
        pltpu.VMEM((128, D), jnp.float32),
        pltpu.VMEM((NP1 * D,), jnp.float32),
        pltpu.VMEM((NP1 * 16,), jnp.float32),
        pltpu.SemaphoreType.DMA,
        pltpu.SemaphoreType.DMA,
    ],
)

_BOUNDS = np.concatenate([
    np.arange(NSUB + 1, dtype=np.int32) * NP0,
    C0TOT + np.arange(1, NSUB + 1, dtype=np.int32) * NP1,
])
_LIN_OF_WID = np.array([(w % 2) * NSUB + w // 2 for w in range(2 * NSUB)],
                       dtype=np.int32)


def _edge_meta(rows):
    starts = jnp.searchsorted(rows, jnp.asarray(_BOUNDS)).astype(jnp.int32)
    se = jnp.zeros((2 * NSUB, 16), jnp.int32)
    se = se.at[:, 0].set(starts[_LIN_OF_WID])
    se = se.at[:, 1].set(starts[_LIN_OF_WID + 1])
    return se


def _pad_edges(cols, rows, w):
    return (jnp.pad(cols, (0, CAP)), jnp.pad(rows, (0, CAP)),
            jnp.pad(w, (0, CAP)))


BLKR = 512


def _mm_body(x_ref, num_ref, den_ref, wa_ref, wb_ref, o_ref):
    den = jnp.sum(den_ref[...], axis=1, keepdims=True)
    den = jnp.where(den > 0.0, den, 1.0)
    neib = num_ref[...] / den
    acc = jnp.dot(x_ref[...], wa_ref[...], preferred_element_type=jnp.float32)
    acc += jnp.dot(neib, wb_ref[...], preferred_element_type=jnp.float32)
    o_ref[...] = jnp.maximum(acc, 0.0)


def _dense(x, num, den, W):
    h = W.shape[1]
    wa, wb = W[:D], W[D:]
    return pl.pallas_call(
        _mm_body,
        grid=(NPAD // BLKR,),
        in_specs=[
            pl.BlockSpec((BLKR, D), lambda i: (i, 0)),
            pl.BlockSpec((BLKR, D), lambda i: (i, 0)),
            pl.BlockSpec((BLKR, 16), lambda i: (i, 0)),
            pl.BlockSpec((D, h), lambda i: (0, 0)),
            pl.BlockSpec((D, h), lambda i: (0, 0)),
        ],
        out_specs=pl.BlockSpec((BLKR, h), lambda i: (i, 0)),
        out_shape=jax.ShapeDtypeStruct((NPAD, h), jnp.float32),
    )(x, num, den, wa, wb)


def _layer(x, W, cols, rows, w, se):
    num, den = _agg(x, cols, rows, w, se)
    return _dense(x, num.reshape(NPAD, D), den.reshape(NPAD, 16), W)


def kernel(raw_features, W1, W2, w1, w2, rows1, cols1, rows2, cols2):
    n = raw_features.shape[1]
    feat = jnp.zeros((NPAD, D), jnp.float32).at[:n].set(raw_features[0])
    e1 = _pad_edges(cols1, rows1, w1)
    e2 = _pad_edges(cols2, rows2, w2)
    se1 = _edge_meta(rows1)
    se2 = _edge_meta(rows2)

    h1 = _layer(feat, W1, *e1, se1)
    h2 = _layer(h1, W2, *e2, se2)
    return h2[:n][None]


# speedup vs baseline: 1.2783x; 1.2783x over previous
import jax
import jax.numpy as jnp
from jax import lax
from jax.experimental import pallas as pl
from jax.experimental.pallas import tpu as pltpu
from jax.experimental.pallas import tpu_sc as plsc

NTILE = 32
NP = 320
NPAD = NTILE * NP
MAXDEG = 7
NB = -(-(MAXDEG * NP + 8) // 128)
CAP = NB * 128
D = 128
LANES = 8


def _bcast_lane(v, l):
    idx = jnp.full((16, 1), l, jnp.int32)
    dnums = lax.GatherDimensionNumbers(
        offset_dims=(), collapsed_slice_dims=(0,), start_index_map=(0,))
    return lax.gather(v, idx, dnums, (1,),
                      mode=lax.GatherScatterMode.PROMISE_IN_BOUNDS)


def _agg_body(feat, cols_p, rows_p, w_p, se, num_out, den_out,
              colsb, rowsb, wb, sev, gath0, gath1, num_acc, den_acc,
              sem0, sem1):
    c = lax.axis_index("c")
    s = lax.axis_index("s")
    wid = s * 2 + c

    pltpu.sync_copy(se.at[wid], sev)
    sevv = sev[...]
    start = sevv[0]
    end = sevv[1]
    astart = pl.multiple_of(jnp.bitwise_and(start, -8), 8)
    lo = start - astart
    hi = lo + (end - start)

    pltpu.sync_copy(cols_p.at[pl.ds(astart, CAP)], colsb)
    pltpu.sync_copy(rows_p.at[pl.ds(astart, CAP)], rowsb)
    pltpu.sync_copy(w_p.at[pl.ds(astart, CAP)], wb)

    zero = jnp.zeros((16,), jnp.float32)

    def zrow(i, _):
        for u in range(16):
            num_acc[pl.ds(pl.multiple_of(i * 256 + u * 16, 16), 16)] = zero
        return 0

    lax.fori_loop(0, NP * D // 256, zrow, 0)

    def zden(i, _):
        for u in range(16):
            den_acc[pl.ds(pl.multiple_of(i * 256 + u * 16, 16), 16)] = zero
        return 0

    lax.fori_loop(0, NP * 16 // 256, zden, 0)

    iota = lax.broadcasted_iota(jnp.int32, (16,), 0)
    node_base = wid * NP

    def block_slice(j):
        return colsb.at[pl.ds(pl.multiple_of(j * 128, 128), 128)]

    def process_block(j, gath):
        def gbody(g, _):
            base = pl.multiple_of(g * 16, 16)
            sbase = j * 128 + base
            w_v = wb[pl.ds(sbase, 16)]
            r_v = rowsb[pl.ds(sbase, 16)]
            gidx = sbase + iota
            valid = (gidx >= lo) & (gidx < hi)
            wm_v = jnp.where(valid, w_v, 0.0)
            rl_v = jnp.clip(r_v - node_base, 0, NP - 1)
            plsc.addupdate_scatter(den_acc, [rl_v * 16 + iota], wm_v)
            ribase = rl_v * D
            for c0 in range(0, 16, 4):
                scaled = []
                for l in range(c0, c0 + 4):
                    w_b = _bcast_lane(wm_v, l)
                    ri = _bcast_lane(ribase, l) + iota
                    e = base + l
                    for k in range(LANES):
                        v = gath[e, pl.ds(k * 16, 16)]
                        scaled.append((ri + k * 16, v * w_b))
                for idx, val in scaled:
                    plsc.addupdate_scatter(num_acc, [idx], val)
            return 0

        lax.fori_loop(0, 128 // 16, gbody, 0)

    pltpu.async_copy(feat.at[block_slice(0)], gath0, sem0)

    def pair_body(t, _):
        j0 = t * 2
        j1 = j0 + 1
        pltpu.make_async_copy(feat.at[block_slice(j0)], gath0, sem0).wait()
        pltpu.async_copy(feat.at[block_slice(j1)], gath1, sem1)
        process_block(j0, gath0)
        pltpu.make_async_copy(feat.at[block_slice(j1)], gath1, sem1).wait()

        @pl.when(t < NB // 2 - 1)
        def _():
            pltpu.async_copy(feat.at[block_slice(j0 + 2)], gath0, sem0)

        process_block(j1, gath1)
        return 0

    lax.fori_loop(0, NB // 2, pair_body, 0)

    pltpu.sync_copy(num_acc, num_out.at[pl.ds(wid * NP * D, NP * D)])
    pltpu.sync_copy(den_acc, den_out.at[pl.ds(wid * NP * 16, NP * 16)])


_agg = pl.kernel(
    _agg_body,
    out_type=[jax.ShapeDtypeStruct((NPAD * D,), jnp.float32),
              jax.ShapeDtypeStruct((NPAD * 16,), jnp.float32)],
    mesh=plsc.VectorSubcoreMesh(core_axis_name="c", subcore_axis_name="s"),
    compiler_params=pltpu.CompilerParams(needs_layout_passes=False),
    scratch_types=[
        pltpu.VMEM((CAP,), jnp.int32),
        pltpu.VMEM((CAP,), jnp.int32),
        pltpu.VMEM((CAP,), jnp.float32),
        pltpu.VMEM((16,), jnp.int32),
        pltpu.VMEM((128, D), jnp.float32),
        pltpu.VMEM((128, D), jnp.float32),
        pltpu.VMEM((NP * D,), jnp.float32),
        pltpu.VMEM((NP * 16,), jnp.float32),
        pltpu.SemaphoreType.DMA,
        pltpu.SemaphoreType.DMA,
    ],
)


def _edge_meta(rows):
    bounds = jnp.arange(NTILE + 1, dtype=jnp.int32) * NP
    starts = jnp.searchsorted(rows, bounds).astype(jnp.int32)
    se = jnp.zeros((NTILE, 16), jnp.int32)
    se = se.at[:, 0].set(starts[:NTILE]).at[:, 1].set(starts[1:])
    return se


def _pad_edges(cols, rows, w):
    return (jnp.pad(cols, (0, CAP)), jnp.pad(rows, (0, CAP)),
            jnp.pad(w, (0, CAP)))


BLKR = 512


def _mm_body(x_ref, num_ref, den_ref, wa_ref, wb_ref, o_ref):
    den = jnp.sum(den_ref[...], axis=1, keepdims=True)
    den = jnp.where(den > 0.0, den, 1.0)
    neib = num_ref[...] / den
    acc = jnp.dot(x_ref[...], wa_ref[...], preferred_element_type=jnp.float32)
    acc += jnp.dot(neib, wb_ref[...], preferred_element_type=jnp.float32)
    o_ref[...] = jnp.maximum(acc, 0.0)


def _dense(x, num, den, W):
    h = W.shape[1]
    wa, wb = W[:D], W[D:]
    return pl.pallas_call(
        _mm_body,
        grid=(NPAD // BLKR,),
        in_specs=[
            pl.BlockSpec((BLKR, D), lambda i: (i, 0)),
            pl.BlockSpec((BLKR, D), lambda i: (i, 0)),
            pl.BlockSpec((BLKR, 16), lambda i: (i, 0)),
            pl.BlockSpec((D, h), lambda i: (0, 0)),
            pl.BlockSpec((D, h), lambda i: (0, 0)),
        ],
        out_specs=pl.BlockSpec((BLKR, h), lambda i: (i, 0)),
        out_shape=jax.ShapeDtypeStruct((NPAD, h), jnp.float32),
    )(x, num, den, wa, wb)


def _layer(x, W, cols, rows, w, se):
    num, den = _agg(x, cols, rows, w, se)
    return _dense(x, num.reshape(NPAD, D), den.reshape(NPAD, 16), W)


def kernel(raw_features, W1, W2, w1, w2, rows1, cols1, rows2, cols2):
    n = raw_features.shape[1]
    feat = jnp.zeros((NPAD, D), jnp.float32).at[:n].set(raw_features[0])
    e1 = _pad_edges(cols1, rows1, w1)
    e2 = _pad_edges(cols2, rows2, w2)
    se1 = _edge_meta(rows1)
    se2 = _edge_meta(rows2)

    h1 = _layer(feat, W1, *e1, se1)
    h2 = _layer(h1, W2, *e2, se2)
    return h2[:n][None]
